# Initial kernel scaffold; baseline (speedup 1.0000x reference)
import jax
import jax.numpy as jnp
from jax.experimental import pallas as pl


def kernel(e, p, edge_index, params):
    raise NotImplementedError("write your pallas kernel here")


# speedup vs baseline: 2.2272x; 2.2272x over previous
import functools

import jax
import jax.numpy as jnp
from jax import lax
from jax.experimental import pallas as pl
from jax.experimental.pallas import tpu as pltpu
from jax.experimental.pallas import tpu_sc as plsc

N = 10000
E = 160000
D = 256
H = 128
NS = 16
NC = 2
ET = E // NS
K = 80
NCH = ET // K
NPT = 624
ZR = 16


BE = 2000
BN = 2000


def _m0c_body(e_ref, p_ref, s0, t0, W0, b0, Wfe, wp, bc,
              f_ref, c1, c2, c3):
    x = e_ref[...] * s0[...] + t0[...]
    f = jnp.maximum(
        jnp.dot(x, W0[...], preferred_element_type=jnp.float32) + b0[...], 0.0)
    call = (jnp.dot(f, Wfe[...], preferred_element_type=jnp.float32)
            + p_ref[...] * wp[...] + bc[...])
    f_ref[0] = f[:, :H]
    f_ref[1] = f[:, H:]
    for i, cr in enumerate((c1, c2, c3)):
        cr[0] = call[:, i * D:i * D + H]
        cr[1] = call[:, i * D + H:(i + 1) * D]


def _m0c(e, p, s0, t0, W0, b0, Wfe, wp, bc):
    eh = jax.ShapeDtypeStruct((NC, E, H), jnp.float32)
    full = lambda shape: pl.BlockSpec(shape, lambda i: (0, 0))
    return pl.pallas_call(
        _m0c_body,
        grid=(E // BE,),
        in_specs=[
            pl.BlockSpec((BE, D), lambda i: (i, 0)),
            pl.BlockSpec((BE, 1), lambda i: (i, 0)),
            full((1, D)), full((1, D)), full((D, D)), full((1, D)),
            full((D, 3 * D)), full((1, 3 * D)), full((1, 3 * D)),
        ],
        out_specs=[pl.BlockSpec((NC, BE, H), lambda i: (0, i, 0))] * 4,
        out_shape=[eh, eh, eh, eh],
    )(e, p, s0, t0, W0, b0, Wfe, wp, bc)


def _ma_body(s_ref, cnt_ref, Wab, t_ref):
    h = jnp.concatenate([s_ref[0], s_ref[1]], axis=1)
    cm = jnp.maximum(cnt_ref[:, 0:1], 1.0)
    ab = jnp.dot(h / cm, Wab[...], preferred_element_type=jnp.float32)
    t_ref[0] = ab[:, 0 * H:1 * H]
    t_ref[1] = ab[:, 1 * H:2 * H]
    t_ref[2] = ab[:, 2 * H:3 * H]
    t_ref[3] = ab[:, 3 * H:4 * H]


def _ma(sums, cnt, Wab):
    return pl.pallas_call(
        _ma_body,
        grid=(N // BN,),
        in_specs=[
            pl.BlockSpec((NC, BN, H), lambda i: (0, i, 0)),
            pl.BlockSpec((BN, H), lambda i: (i, 0)),
            pl.BlockSpec((D, 4 * H), lambda i: (0, 0)),
        ],
        out_specs=pl.BlockSpec((4, BN, H), lambda i: (0, i, 0)),
        out_shape=jax.ShapeDtypeStruct((4, N, H), jnp.float32),
    )(sums, cnt, Wab)


def _mf_body(fs_ref, hs_ref, cnt_ref, Wo, bo, y_ref):
    cm = jnp.maximum(cnt_ref[:, 0:1], 1.0)
    fn = jnp.concatenate([fs_ref[0], fs_ref[1]], axis=1) / cm
    hn = jnp.concatenate([hs_ref[0], hs_ref[1]], axis=1) / cm
    x = jnp.concatenate([fn, hn], axis=1)
    y_ref[...] = jnp.maximum(
        jnp.dot(x, Wo[...], preferred_element_type=jnp.float32) + bo[...], 0.0)


def _mf(fsums, hsums, cnt, Wo, bo):
    return pl.pallas_call(
        _mf_body,
        grid=(N // BN,),
        in_specs=[
            pl.BlockSpec((NC, BN, H), lambda i: (0, i, 0)),
            pl.BlockSpec((NC, BN, H), lambda i: (0, i, 0)),
            pl.BlockSpec((BN, H), lambda i: (i, 0)),
            pl.BlockSpec((2 * D, D), lambda i: (0, 0)),
            pl.BlockSpec((1, D), lambda i: (0, 0)),
        ],
        out_specs=pl.BlockSpec((BN, D), lambda i: (i, 0)),
        out_shape=jax.ShapeDtypeStruct((N, D), jnp.float32),
    )(fsums, hsums, cnt, Wo, bo)


_MESH = plsc.VectorSubcoreMesh(core_axis_name="c", subcore_axis_name="s")


def _fill(ref, rows, value):
    width = ref.shape[1]
    vec = jnp.full((16,), value, jnp.float32)

    def body(r, _):
        for v in range(width // 16):
            ref[r, pl.ds(v * 16, 16)] = vec
        return 0

    lax.fori_loop(0, rows, body, 0)


def _zero_acc(acc, zb, s):
    for j in range(NPT // ZR):
        pltpu.sync_copy(zb, acc.at[pl.ds(s * NPT + j * ZR, ZR)])

    @pl.when(s == NS - 1)
    def _():
        pltpu.sync_copy(zb, acc.at[pl.ds(NS * NPT, 16)])


def _write_rows(src_spm, dst_hbm, s, dst_base):
    pltpu.sync_copy(src_spm.at[pl.ds(s * NPT, NPT)],
                    dst_hbm.at[pl.ds(dst_base + s * NPT, NPT)])

    @pl.when(s == NS - 1)
    def _():
        pltpu.sync_copy(src_spm.at[pl.ds(NS * NPT, 16)],
                        dst_hbm.at[pl.ds(dst_base + NS * NPT, 16)])


@functools.partial(
    pl.kernel,
    mesh=_MESH,
    out_type=jax.ShapeDtypeStruct((NC * N, H), jnp.float32),
    scratch_types=[
        pltpu.VMEM_SHARED((N, H), jnp.float32),
        pltpu.VMEM((K, H), jnp.float32),
        pltpu.VMEM((K,), jnp.int32),
        pltpu.VMEM((ZR, H), jnp.float32),
    ],
)
def _s0(f_hbm, dst_hbm, sums_out, acc, vbuf, ib, zb):
    c = lax.axis_index("c")
    s = lax.axis_index("s")
    _fill(zb, ZR, 0.0)
    _zero_acc(acc, zb, s)
    plsc.subcore_barrier()

    def chunk(k, _):
        off = s * ET + k * K
        pltpu.sync_copy(dst_hbm.at[pl.ds(off, K)], ib)
        pltpu.sync_copy(f_hbm.at[pl.ds(c * E + off, K)], vbuf)
        pltpu.sync_copy(vbuf, acc.at[ib], add=True)
        return 0

    lax.fori_loop(0, NCH, chunk, 0)
    plsc.subcore_barrier()
    _write_rows(acc, sums_out, s, c * N)


@functools.partial(
    pl.kernel,
    mesh=_MESH,
    out_type=jax.ShapeDtypeStruct((N, H), jnp.float32),
    scratch_types=[
        pltpu.VMEM_SHARED((N, H), jnp.float32),
        pltpu.VMEM((K,), jnp.int32),
        pltpu.VMEM((K, H), jnp.float32),
        pltpu.VMEM((ZR, H), jnp.float32),
    ],
)
def _cnt(dst_hbm, cnt_out, cacc, ib, ones, zc):
    c = lax.axis_index("c")
    s = lax.axis_index("s")
    _fill(zc, ZR, 0.0)
    _fill(ones, K, 1.0)
    for j in range(NPT // ZR):
        pltpu.sync_copy(zc, cacc.at[pl.ds(s * NPT + j * ZR, ZR)])

    @pl.when(s == NS - 1)
    def _():
        pltpu.sync_copy(zc, cacc.at[pl.ds(NS * NPT, 16)])

    plsc.subcore_barrier()

    def chunk(k, _):
        off = s * ET + k * K
        pltpu.sync_copy(dst_hbm.at[pl.ds(off, K)], ib)
        pltpu.sync_copy(ones, cacc.at[ib], add=True)
        return 0

    lax.fori_loop(0, NCH, chunk, 0)
    plsc.subcore_barrier()

    @pl.when(c == 0)
    def _():
        _write_rows(cacc, cnt_out, s, 0)


@functools.partial(
    pl.kernel,
    mesh=_MESH,
    out_type=jax.ShapeDtypeStruct((NC * N, H), jnp.float32),
    scratch_types=[
        pltpu.VMEM_SHARED((N, H), jnp.float32),
        pltpu.VMEM((K, H), jnp.float32),
        pltpu.VMEM((K, H), jnp.float32),
        pltpu.VMEM((K, H), jnp.float32),
        pltpu.VMEM((K,), jnp.int32),
        pltpu.VMEM((K,), jnp.int32),
        pltpu.VMEM((K,), jnp.int32),
        pltpu.VMEM((ZR, H), jnp.float32),
    ],
)
def _edge(t_hbm, c_hbm, gidx_hbm, dst_hbm, sums_out,
          acc, abuf, bbuf, cbuf, aib, bib, dib, zb):
    c = lax.axis_index("c")
    s = lax.axis_index("s")
    _fill(zb, ZR, 0.0)
    _zero_acc(acc, zb, s)
    plsc.subcore_barrier()

    def chunk(k, _):
        off = s * ET + k * K
        pltpu.sync_copy(gidx_hbm.at[pl.ds(2 * c * E + off, K)], aib)
        pltpu.sync_copy(gidx_hbm.at[pl.ds((2 * c + 1) * E + off, K)], bib)
        pltpu.sync_copy(dst_hbm.at[pl.ds(off, K)], dib)
        pltpu.sync_copy(t_hbm.at[aib], abuf)
        pltpu.sync_copy(t_hbm.at[bib], bbuf)
        pltpu.sync_copy(c_hbm.at[pl.ds(c * E + off, K)], cbuf)

        def row(r, _):
            for v in range(H // 16):
                sl = pl.ds(v * 16, 16)
                cbuf[r, sl] = jnp.maximum(
                    abuf[r, sl] + bbuf[r, sl] + cbuf[r, sl], 0.0)
            return 0

        lax.fori_loop(0, K, row, 0)
        pltpu.sync_copy(cbuf, acc.at[dib], add=True)
        return 0

    lax.fori_loop(0, NCH, chunk, 0)
    plsc.subcore_barrier()
    _write_rows(acc, sums_out, s, c * N)


def _fold(pr):
    s = pr['bn']['gamma'] * lax.rsqrt(pr['bn']['var'] + 1e-5)
    t = pr['bn']['beta'] - pr['bn']['mean'] * s
    Wf = s[:, None] * pr['W']
    bf = t @ pr['W'] + pr['b']
    return Wf, bf


def kernel(e, p, edge_index, params):
    src = edge_index[0].astype(jnp.int32)
    dst = edge_index[1].astype(jnp.int32)
    gidx = jnp.concatenate(
        [src, dst + 2 * N, src + N, dst + 3 * N])

    s0 = params['init']['bn']['gamma'] * lax.rsqrt(params['init']['bn']['var'] + 1e-5)
    t0 = params['init']['bn']['beta'] - params['init']['bn']['mean'] * s0
    W0 = params['init']['W']
    b0 = params['init']['b']

    Wfs, bfs = [], []
    for lp in params['layers']:
        Wf, bf = _fold(lp)
        Wfs.append(Wf)
        bfs.append(bf)
    WoF, boF = _fold(params['out'])

    Wfe = jnp.concatenate([Wf[D:2 * D] for Wf in Wfs], axis=1)
    wp = jnp.concatenate([Wf[2 * D:2 * D + 1] for Wf in Wfs], axis=1)
    bc = jnp.concatenate([bf[None, :] for bf in bfs], axis=1)

    f, c1, c2, c3 = _m0c(
        e, p,
        s0[None, :], t0[None, :], W0, b0[None, :],
        Wfe, wp, bc,
    )

    fsums = _s0(f.reshape(NC * E, H), dst)
    cnt = _cnt(dst)
    fsums3 = fsums.reshape(NC, N, H)

    sums3 = fsums3
    for i, cc in enumerate((c1, c2, c3)):
        Wab = jnp.concatenate([Wfs[i][:D], Wfs[i][2 * D + 1:]], axis=1)
        t = _ma(sums3, cnt, Wab)
        sums = _edge(t.reshape(4 * N, H), cc.reshape(NC * E, H), gidx, dst)
        sums3 = sums.reshape(NC, N, H)

    return _mf(fsums3, sums3, cnt, WoF, boF[None, :])


# speedup vs baseline: 2.6344x; 1.1828x over previous
import functools

import jax
import jax.numpy as jnp
from jax import lax
from jax.experimental import pallas as pl
from jax.experimental.pallas import tpu as pltpu
from jax.experimental.pallas import tpu_sc as plsc

N = 10000
E = 160000
D = 256
H = 128
NS = 16
NC = 2
ET = E // NS
K = 80
NCH = ET // K
NPT = 624
ZR = 16


BE = 2000
BN = 2000


def _m0c_body(e_ref, p_ref, s0, t0, W0, b0, Wfe, wp, bc,
              f_ref, c1, c2, c3):
    x = e_ref[...] * s0[...] + t0[...]
    f = jnp.maximum(
        jnp.dot(x, W0[...], preferred_element_type=jnp.float32) + b0[...], 0.0)
    call = (jnp.dot(f, Wfe[...], preferred_element_type=jnp.float32)
            + p_ref[...] * wp[...] + bc[...])
    f_ref[0] = f[:, :H]
    f_ref[1] = f[:, H:]
    for i, cr in enumerate((c1, c2, c3)):
        cr[0] = call[:, i * D:i * D + H]
        cr[1] = call[:, i * D + H:(i + 1) * D]


def _m0c(e, p, s0, t0, W0, b0, Wfe, wp, bc):
    eh = jax.ShapeDtypeStruct((NC, E, H), jnp.float32)
    full = lambda shape: pl.BlockSpec(shape, lambda i: (0, 0))
    return pl.pallas_call(
        _m0c_body,
        grid=(E // BE,),
        in_specs=[
            pl.BlockSpec((BE, D), lambda i: (i, 0)),
            pl.BlockSpec((BE, 1), lambda i: (i, 0)),
            full((1, D)), full((1, D)), full((D, D)), full((1, D)),
            full((D, 3 * D)), full((1, 3 * D)), full((1, 3 * D)),
        ],
        out_specs=[pl.BlockSpec((NC, BE, H), lambda i: (0, i, 0))] * 4,
        out_shape=[eh, eh, eh, eh],
    )(e, p, s0, t0, W0, b0, Wfe, wp, bc)


def _ma_body(s_ref, cnt_ref, Wab, t_ref):
    h = jnp.concatenate([s_ref[0], s_ref[1]], axis=1)
    cm = jnp.maximum(cnt_ref[:, 0:1], 1.0)
    ab = jnp.dot(h / cm, Wab[...], preferred_element_type=jnp.float32)
    t_ref[0] = ab[:, 0 * H:1 * H]
    t_ref[1] = ab[:, 1 * H:2 * H]
    t_ref[2] = ab[:, 2 * H:3 * H]
    t_ref[3] = ab[:, 3 * H:4 * H]


def _ma(sums, cnt, Wab):
    return pl.pallas_call(
        _ma_body,
        grid=(N // BN,),
        in_specs=[
            pl.BlockSpec((NC, BN, H), lambda i: (0, i, 0)),
            pl.BlockSpec((BN, H), lambda i: (i, 0)),
            pl.BlockSpec((D, 4 * H), lambda i: (0, 0)),
        ],
        out_specs=pl.BlockSpec((4, BN, H), lambda i: (0, i, 0)),
        out_shape=jax.ShapeDtypeStruct((4, N, H), jnp.float32),
    )(sums, cnt, Wab)


def _mf_body(fs_ref, hs_ref, cnt_ref, Wo, bo, y_ref):
    cm = jnp.maximum(cnt_ref[:, 0:1], 1.0)
    fn = jnp.concatenate([fs_ref[0], fs_ref[1]], axis=1) / cm
    hn = jnp.concatenate([hs_ref[0], hs_ref[1]], axis=1) / cm
    x = jnp.concatenate([fn, hn], axis=1)
    y_ref[...] = jnp.maximum(
        jnp.dot(x, Wo[...], preferred_element_type=jnp.float32) + bo[...], 0.0)


def _mf(fsums, hsums, cnt, Wo, bo):
    return pl.pallas_call(
        _mf_body,
        grid=(N // BN,),
        in_specs=[
            pl.BlockSpec((NC, BN, H), lambda i: (0, i, 0)),
            pl.BlockSpec((NC, BN, H), lambda i: (0, i, 0)),
            pl.BlockSpec((BN, H), lambda i: (i, 0)),
            pl.BlockSpec((2 * D, D), lambda i: (0, 0)),
            pl.BlockSpec((1, D), lambda i: (0, 0)),
        ],
        out_specs=pl.BlockSpec((BN, D), lambda i: (i, 0)),
        out_shape=jax.ShapeDtypeStruct((N, D), jnp.float32),
    )(fsums, hsums, cnt, Wo, bo)


_MESH = plsc.VectorSubcoreMesh(core_axis_name="c", subcore_axis_name="s")


def _fill(ref, rows, value):
    width = ref.shape[1]
    vec = jnp.full((16,), value, jnp.float32)

    def body(r, _):
        for v in range(width // 16):
            ref[r, pl.ds(v * 16, 16)] = vec
        return 0

    lax.fori_loop(0, rows, body, 0)


def _zero_acc(acc, zb, s):
    for j in range(NPT // ZR):
        pltpu.sync_copy(zb, acc.at[pl.ds(s * NPT + j * ZR, ZR)])

    @pl.when(s == NS - 1)
    def _():
        pltpu.sync_copy(zb, acc.at[pl.ds(NS * NPT, 16)])


def _write_rows(src_spm, dst_hbm, s, dst_base):
    pltpu.sync_copy(src_spm.at[pl.ds(s * NPT, NPT)],
                    dst_hbm.at[pl.ds(dst_base + s * NPT, NPT)])

    @pl.when(s == NS - 1)
    def _():
        pltpu.sync_copy(src_spm.at[pl.ds(NS * NPT, 16)],
                        dst_hbm.at[pl.ds(dst_base + NS * NPT, 16)])


@functools.partial(
    pl.kernel,
    mesh=_MESH,
    out_type=jax.ShapeDtypeStruct((NC * N, H), jnp.float32),
    scratch_types=[
        pltpu.VMEM_SHARED((N, H), jnp.float32),
        pltpu.VMEM((K, H), jnp.float32),
        pltpu.VMEM((K,), jnp.int32),
        pltpu.VMEM((ZR, H), jnp.float32),
    ],
)
def _s0(f_hbm, dst_hbm, sums_out, acc, vbuf, ib, zb):
    c = lax.axis_index("c")
    s = lax.axis_index("s")
    _fill(zb, ZR, 0.0)
    _zero_acc(acc, zb, s)
    plsc.subcore_barrier()

    def chunk(k, _):
        off = s * ET + k * K
        pltpu.sync_copy(dst_hbm.at[pl.ds(off, K)], ib)
        pltpu.sync_copy(f_hbm.at[pl.ds(c * E + off, K)], vbuf)
        pltpu.sync_copy(vbuf, acc.at[ib], add=True)
        return 0

    lax.fori_loop(0, NCH, chunk, 0)
    plsc.subcore_barrier()
    _write_rows(acc, sums_out, s, c * N)


@functools.partial(
    pl.kernel,
    mesh=_MESH,
    out_type=jax.ShapeDtypeStruct((N, H), jnp.float32),
    scratch_types=[
        pltpu.VMEM_SHARED((N, H), jnp.float32),
        pltpu.VMEM((K,), jnp.int32),
        pltpu.VMEM((K, H), jnp.float32),
        pltpu.VMEM((ZR, H), jnp.float32),
    ],
)
def _cnt(dst_hbm, cnt_out, cacc, ib, ones, zc):
    c = lax.axis_index("c")
    s = lax.axis_index("s")
    _fill(zc, ZR, 0.0)
    _fill(ones, K, 1.0)
    for j in range(NPT // ZR):
        pltpu.sync_copy(zc, cacc.at[pl.ds(s * NPT + j * ZR, ZR)])

    @pl.when(s == NS - 1)
    def _():
        pltpu.sync_copy(zc, cacc.at[pl.ds(NS * NPT, 16)])

    plsc.subcore_barrier()

    def chunk(k, _):
        off = s * ET + k * K
        pltpu.sync_copy(dst_hbm.at[pl.ds(off, K)], ib)
        pltpu.sync_copy(ones, cacc.at[ib], add=True)
        return 0

    lax.fori_loop(0, NCH, chunk, 0)
    plsc.subcore_barrier()

    @pl.when(c == 0)
    def _():
        _write_rows(cacc, cnt_out, s, 0)


@functools.partial(
    pl.kernel,
    mesh=_MESH,
    out_type=jax.ShapeDtypeStruct((NC * N, H), jnp.float32),
    scratch_types=[
        pltpu.VMEM_SHARED((N, H), jnp.float32),
        pltpu.VMEM((K, H), jnp.float32),
        pltpu.VMEM((K, H), jnp.float32),
        pltpu.VMEM((K, H), jnp.float32),
        pltpu.VMEM((K,), jnp.int32),
        pltpu.VMEM((K,), jnp.int32),
        pltpu.VMEM((K,), jnp.int32),
        pltpu.VMEM((ZR, H), jnp.float32),
        pltpu.SemaphoreType.DMA,
        pltpu.SemaphoreType.DMA,
        pltpu.SemaphoreType.DMA,
    ],
)
def _edge(t_hbm, c_hbm, gidx_hbm, dst_hbm, sums_out,
          acc, abuf, bbuf, cbuf, aib, bib, dib, zb, sa, sb, sc):
    c = lax.axis_index("c")
    s = lax.axis_index("s")
    _fill(zb, ZR, 0.0)
    _zero_acc(acc, zb, s)
    plsc.subcore_barrier()

    def chunk(k, _):
        off = s * ET + k * K
        pltpu.sync_copy(gidx_hbm.at[pl.ds(2 * c * E + off, K)], aib)
        pltpu.sync_copy(gidx_hbm.at[pl.ds((2 * c + 1) * E + off, K)], bib)
        pltpu.sync_copy(dst_hbm.at[pl.ds(off, K)], dib)
        da = pltpu.async_copy(t_hbm.at[aib], abuf, sa)
        db = pltpu.async_copy(t_hbm.at[bib], bbuf, sb)
        dc = pltpu.async_copy(c_hbm.at[pl.ds(c * E + off, K)], cbuf, sc)
        da.wait()
        db.wait()
        dc.wait()

        def row(r, _):
            for v in range(H // 16):
                sl = pl.ds(v * 16, 16)
                cbuf[r, sl] = jnp.maximum(
                    abuf[r, sl] + bbuf[r, sl] + cbuf[r, sl], 0.0)
            return 0

        lax.fori_loop(0, K, row, 0)
        pltpu.sync_copy(cbuf, acc.at[dib], add=True)
        return 0

    lax.fori_loop(0, NCH, chunk, 0)
    plsc.subcore_barrier()
    _write_rows(acc, sums_out, s, c * N)


def _fold(pr):
    s = pr['bn']['gamma'] * lax.rsqrt(pr['bn']['var'] + 1e-5)
    t = pr['bn']['beta'] - pr['bn']['mean'] * s
    Wf = s[:, None] * pr['W']
    bf = t @ pr['W'] + pr['b']
    return Wf, bf


def kernel(e, p, edge_index, params):
    src = edge_index[0].astype(jnp.int32)
    dst = edge_index[1].astype(jnp.int32)
    gidx = jnp.concatenate(
        [src, dst + 2 * N, src + N, dst + 3 * N])

    s0 = params['init']['bn']['gamma'] * lax.rsqrt(params['init']['bn']['var'] + 1e-5)
    t0 = params['init']['bn']['beta'] - params['init']['bn']['mean'] * s0
    W0 = params['init']['W']
    b0 = params['init']['b']

    Wfs, bfs = [], []
    for lp in params['layers']:
        Wf, bf = _fold(lp)
        Wfs.append(Wf)
        bfs.append(bf)
    WoF, boF = _fold(params['out'])

    Wfe = jnp.concatenate([Wf[D:2 * D] for Wf in Wfs], axis=1)
    wp = jnp.concatenate([Wf[2 * D:2 * D + 1] for Wf in Wfs], axis=1)
    bc = jnp.concatenate([bf[None, :] for bf in bfs], axis=1)

    f, c1, c2, c3 = _m0c(
        e, p,
        s0[None, :], t0[None, :], W0, b0[None, :],
        Wfe, wp, bc,
    )

    fsums = _s0(f.reshape(NC * E, H), dst)
    cnt = _cnt(dst)
    fsums3 = fsums.reshape(NC, N, H)

    sums3 = fsums3
    for i, cc in enumerate((c1, c2, c3)):
        Wab = jnp.concatenate([Wfs[i][:D], Wfs[i][2 * D + 1:]], axis=1)
        t = _ma(sums3, cnt, Wab)
        sums = _edge(t.reshape(4 * N, H), cc.reshape(NC * E, H), gidx, dst)
        sums3 = sums.reshape(NC, N, H)

    return _mf(fsums3, sums3, cnt, WoF, boF[None, :])


# speedup vs baseline: 2.9210x; 1.1088x over previous
import functools

import jax
import jax.numpy as jnp
from jax import lax
from jax.experimental import pallas as pl
from jax.experimental.pallas import tpu as pltpu
from jax.experimental.pallas import tpu_sc as plsc

N = 10000
E = 160000
D = 256
H = 128
NS = 16
NC = 2
ET = E // NS
K = 80
NCH = ET // K
NPT = 624
ZR = 16


BE = 2000
BN = 2000


def _m0c_body(e_ref, p_ref, s0, t0, W0, b0, Wfe, wp, bc,
              f_ref, c1, c2, c3):
    x = e_ref[...] * s0[...] + t0[...]
    f = jnp.maximum(
        jnp.dot(x, W0[...], preferred_element_type=jnp.float32) + b0[...], 0.0)
    call = (jnp.dot(f, Wfe[...], preferred_element_type=jnp.float32)
            + p_ref[...] * wp[...] + bc[...])
    f_ref[0] = f[:, :H]
    f_ref[1] = f[:, H:]
    for i, cr in enumerate((c1, c2, c3)):
        cr[0] = call[:, i * D:i * D + H]
        cr[1] = call[:, i * D + H:(i + 1) * D]


def _m0c(e, p, s0, t0, W0, b0, Wfe, wp, bc):
    eh = jax.ShapeDtypeStruct((NC, E, H), jnp.float32)
    full = lambda shape: pl.BlockSpec(shape, lambda i: (0, 0))
    return pl.pallas_call(
        _m0c_body,
        grid=(E // BE,),
        in_specs=[
            pl.BlockSpec((BE, D), lambda i: (i, 0)),
            pl.BlockSpec((BE, 1), lambda i: (i, 0)),
            full((1, D)), full((1, D)), full((D, D)), full((1, D)),
            full((D, 3 * D)), full((1, 3 * D)), full((1, 3 * D)),
        ],
        out_specs=[pl.BlockSpec((NC, BE, H), lambda i: (0, i, 0))] * 4,
        out_shape=[eh, eh, eh, eh],
    )(e, p, s0, t0, W0, b0, Wfe, wp, bc)


def _ma_body(s_ref, cnt_ref, Wab, t_ref):
    h = jnp.concatenate([s_ref[0], s_ref[1]], axis=1)
    cm = jnp.maximum(cnt_ref[:, 0:1], 1.0)
    ab = jnp.dot(h / cm, Wab[...], preferred_element_type=jnp.float32)
    t_ref[0] = ab[:, 0 * H:1 * H]
    t_ref[1] = ab[:, 1 * H:2 * H]
    t_ref[2] = ab[:, 2 * H:3 * H]
    t_ref[3] = ab[:, 3 * H:4 * H]


def _ma(sums, cnt, Wab):
    return pl.pallas_call(
        _ma_body,
        grid=(N // BN,),
        in_specs=[
            pl.BlockSpec((NC, BN, H), lambda i: (0, i, 0)),
            pl.BlockSpec((BN, H), lambda i: (i, 0)),
            pl.BlockSpec((D, 4 * H), lambda i: (0, 0)),
        ],
        out_specs=pl.BlockSpec((4, BN, H), lambda i: (0, i, 0)),
        out_shape=jax.ShapeDtypeStruct((4, N, H), jnp.float32),
    )(sums, cnt, Wab)


def _mf_body(fs_ref, hs_ref, cnt_ref, Wo, bo, y_ref):
    cm = jnp.maximum(cnt_ref[:, 0:1], 1.0)
    fn = jnp.concatenate([fs_ref[0], fs_ref[1]], axis=1) / cm
    hn = jnp.concatenate([hs_ref[0], hs_ref[1]], axis=1) / cm
    x = jnp.concatenate([fn, hn], axis=1)
    y_ref[...] = jnp.maximum(
        jnp.dot(x, Wo[...], preferred_element_type=jnp.float32) + bo[...], 0.0)


def _mf(fsums, hsums, cnt, Wo, bo):
    return pl.pallas_call(
        _mf_body,
        grid=(N // BN,),
        in_specs=[
            pl.BlockSpec((NC, BN, H), lambda i: (0, i, 0)),
            pl.BlockSpec((NC, BN, H), lambda i: (0, i, 0)),
            pl.BlockSpec((BN, H), lambda i: (i, 0)),
            pl.BlockSpec((2 * D, D), lambda i: (0, 0)),
            pl.BlockSpec((1, D), lambda i: (0, 0)),
        ],
        out_specs=pl.BlockSpec((BN, D), lambda i: (i, 0)),
        out_shape=jax.ShapeDtypeStruct((N, D), jnp.float32),
    )(fsums, hsums, cnt, Wo, bo)


_MESH = plsc.VectorSubcoreMesh(core_axis_name="c", subcore_axis_name="s")


def _fill(ref, rows, value):
    width = ref.shape[1]
    vec = jnp.full((16,), value, jnp.float32)

    def body(r, _):
        for v in range(width // 16):
            ref[r, pl.ds(v * 16, 16)] = vec
        return 0

    lax.fori_loop(0, rows, body, 0)


def _zero_acc(acc, zb, s):
    for j in range(NPT // ZR):
        pltpu.sync_copy(zb, acc.at[pl.ds(s * NPT + j * ZR, ZR)])

    @pl.when(s == NS - 1)
    def _():
        pltpu.sync_copy(zb, acc.at[pl.ds(NS * NPT, 16)])


def _write_rows(src_spm, dst_hbm, s, dst_base):
    pltpu.sync_copy(src_spm.at[pl.ds(s * NPT, NPT)],
                    dst_hbm.at[pl.ds(dst_base + s * NPT, NPT)])

    @pl.when(s == NS - 1)
    def _():
        pltpu.sync_copy(src_spm.at[pl.ds(NS * NPT, 16)],
                        dst_hbm.at[pl.ds(dst_base + NS * NPT, 16)])


@functools.partial(
    pl.kernel,
    mesh=_MESH,
    out_type=jax.ShapeDtypeStruct((NC * N, H), jnp.float32),
    scratch_types=[
        pltpu.VMEM_SHARED((N, H), jnp.float32),
        pltpu.VMEM((K, H), jnp.float32),
        pltpu.VMEM((K,), jnp.int32),
        pltpu.VMEM((ZR, H), jnp.float32),
        pltpu.SemaphoreType.DMA,
    ],
)
def _s0(f_hbm, dst_hbm, sums_out, acc, vbuf, ib, zb, sv):
    c = lax.axis_index("c")
    s = lax.axis_index("s")
    _fill(zb, ZR, 0.0)
    _zero_acc(acc, zb, s)
    plsc.subcore_barrier()

    def chunk(k, _):
        off = s * ET + k * K
        dv = pltpu.async_copy(f_hbm.at[pl.ds(c * E + off, K)], vbuf, sv)
        pltpu.sync_copy(dst_hbm.at[pl.ds(off, K)], ib)
        dv.wait()
        pltpu.sync_copy(vbuf, acc.at[ib], add=True)
        return 0

    lax.fori_loop(0, NCH, chunk, 0)
    plsc.subcore_barrier()
    _write_rows(acc, sums_out, s, c * N)


@functools.partial(
    pl.kernel,
    mesh=_MESH,
    out_type=jax.ShapeDtypeStruct((N, H), jnp.float32),
    scratch_types=[
        pltpu.VMEM_SHARED((N, H), jnp.float32),
        pltpu.VMEM((K,), jnp.int32),
        pltpu.VMEM((K, H), jnp.float32),
        pltpu.VMEM((ZR, H), jnp.float32),
    ],
)
def _cnt(dst_hbm, cnt_out, cacc, ib, ones, zc):
    c = lax.axis_index("c")
    s = lax.axis_index("s")
    _fill(zc, ZR, 0.0)
    _fill(ones, K, 1.0)
    for j in range(NPT // ZR):
        pltpu.sync_copy(zc, cacc.at[pl.ds(s * NPT + j * ZR, ZR)])

    @pl.when(s == NS - 1)
    def _():
        pltpu.sync_copy(zc, cacc.at[pl.ds(NS * NPT, 16)])

    plsc.subcore_barrier()

    def chunk(k, _):
        off = s * ET + k * K
        pltpu.sync_copy(dst_hbm.at[pl.ds(off, K)], ib)
        pltpu.sync_copy(ones, cacc.at[ib], add=True)
        return 0

    lax.fori_loop(0, NCH, chunk, 0)
    plsc.subcore_barrier()

    @pl.when(c == 0)
    def _():
        _write_rows(cacc, cnt_out, s, 0)


K2 = 40
NCH2 = ET // K2


@functools.partial(
    pl.kernel,
    mesh=_MESH,
    out_type=jax.ShapeDtypeStruct((NC * N, H), jnp.float32),
    scratch_types=[
        pltpu.VMEM_SHARED((N, H), jnp.float32),
        pltpu.VMEM((K2, H), jnp.float32),
        pltpu.VMEM((K2, H), jnp.float32),
        pltpu.VMEM((K2, H), jnp.float32),
        pltpu.VMEM((K2, H), jnp.float32),
        pltpu.VMEM((K2, H), jnp.float32),
        pltpu.VMEM((K2, H), jnp.float32),
        pltpu.VMEM((K2,), jnp.int32),
        pltpu.VMEM((K2,), jnp.int32),
        pltpu.VMEM((K2,), jnp.int32),
        pltpu.VMEM((K2,), jnp.int32),
        pltpu.VMEM((K2,), jnp.int32),
        pltpu.VMEM((K2,), jnp.int32),
        pltpu.VMEM((ZR, H), jnp.float32),
        pltpu.SemaphoreType.DMA,
        pltpu.SemaphoreType.DMA,
    ],
)
def _edge(t_hbm, c_hbm, gidx_hbm, dst_hbm, sums_out,
          acc, ab0, ab1, bb0, bb1, cb0, cb1,
          ai0, ai1, bi0, bi1, di0, di1, zb, sp0, sp1):
    c = lax.axis_index("c")
    s = lax.axis_index("s")
    _fill(zb, ZR, 0.0)
    _zero_acc(acc, zb, s)
    plsc.subcore_barrier()

    bufs = ((ab0, bb0, cb0, ai0, bi0, di0, sp0),
            (ab1, bb1, cb1, ai1, bi1, di1, sp1))

    def fire(ck, b):
        abuf, bbuf, cbuf, aib, bib, dib, sp = bufs[b]
        off = s * ET + ck * K2
        pltpu.sync_copy(gidx_hbm.at[pl.ds(2 * c * E + off, K2)], aib)
        pltpu.sync_copy(gidx_hbm.at[pl.ds((2 * c + 1) * E + off, K2)], bib)
        pltpu.sync_copy(dst_hbm.at[pl.ds(off, K2)], dib)
        pltpu.async_copy(t_hbm.at[aib], abuf, sp)
        pltpu.async_copy(t_hbm.at[bib], bbuf, sp)
        pltpu.async_copy(c_hbm.at[pl.ds(c * E + off, K2)], cbuf, sp)

    def drain_compute(b):
        abuf, bbuf, cbuf, aib, bib, dib, sp = bufs[b]
        pltpu.make_async_copy(t_hbm.at[pl.ds(0, K2)], abuf, sp).wait()
        pltpu.make_async_copy(t_hbm.at[pl.ds(0, K2)], bbuf, sp).wait()
        pltpu.make_async_copy(c_hbm.at[pl.ds(0, K2)], cbuf, sp).wait()

        def row(r, _):
            for v in range(H // 16):
                sl = pl.ds(v * 16, 16)
                cbuf[r, sl] = jnp.maximum(
                    abuf[r, sl] + bbuf[r, sl] + cbuf[r, sl], 0.0)
            return 0

        lax.fori_loop(0, K2, row, 0)
        pltpu.sync_copy(cbuf, acc.at[dib], add=True)

    fire(0, 0)
    fire(1, 1)

    def pair(g2, _):
        g = 2 * g2
        for b in range(2):
            drain_compute(b)
            fire(g + 2 + b, b)
        return 0

    lax.fori_loop(0, (NCH2 - 2) // 2, pair, 0)
    drain_compute(0)
    drain_compute(1)
    plsc.subcore_barrier()
    _write_rows(acc, sums_out, s, c * N)


def _fold(pr):
    s = pr['bn']['gamma'] * lax.rsqrt(pr['bn']['var'] + 1e-5)
    t = pr['bn']['beta'] - pr['bn']['mean'] * s
    Wf = s[:, None] * pr['W']
    bf = t @ pr['W'] + pr['b']
    return Wf, bf


def kernel(e, p, edge_index, params):
    src = edge_index[0].astype(jnp.int32)
    dst = edge_index[1].astype(jnp.int32)
    gidx = jnp.concatenate(
        [src, dst + 2 * N, src + N, dst + 3 * N])

    s0 = params['init']['bn']['gamma'] * lax.rsqrt(params['init']['bn']['var'] + 1e-5)
    t0 = params['init']['bn']['beta'] - params['init']['bn']['mean'] * s0
    W0 = params['init']['W']
    b0 = params['init']['b']

    Wfs, bfs = [], []
    for lp in params['layers']:
        Wf, bf = _fold(lp)
        Wfs.append(Wf)
        bfs.append(bf)
    WoF, boF = _fold(params['out'])

    Wfe = jnp.concatenate([Wf[D:2 * D] for Wf in Wfs], axis=1)
    wp = jnp.concatenate([Wf[2 * D:2 * D + 1] for Wf in Wfs], axis=1)
    bc = jnp.concatenate([bf[None, :] for bf in bfs], axis=1)

    f, c1, c2, c3 = _m0c(
        e, p,
        s0[None, :], t0[None, :], W0, b0[None, :],
        Wfe, wp, bc,
    )

    fsums = _s0(f.reshape(NC * E, H), dst)
    cnt = _cnt(dst)
    fsums3 = fsums.reshape(NC, N, H)

    sums3 = fsums3
    for i, cc in enumerate((c1, c2, c3)):
        Wab = jnp.concatenate([Wfs[i][:D], Wfs[i][2 * D + 1:]], axis=1)
        t = _ma(sums3, cnt, Wab)
        sums = _edge(t.reshape(4 * N, H), cc.reshape(NC * E, H), gidx, dst)
        sums3 = sums.reshape(NC, N, H)

    return _mf(fsums3, sums3, cnt, WoF, boF[None, :])
